# baseline (speedup 1.0000x reference)
import numpy as np

import jax
import jax.numpy as jnp
from jax.experimental import pallas as pl
from jax.experimental.pallas import tpu as pltpu

_B, _L, _HID, _NN = 64, 4096, 64, 128
_NEXT = _NN + 1
_ND, _COUT = 32, 10
_HALF = _HID // 2
_MB = 8


def _softmax_rows(x):
    z = x - jnp.max(x, axis=-1, keepdims=True)
    e = jnp.exp(z)
    return e / jnp.sum(e, axis=-1, keepdims=True)


def _ln_lanes(x, g_row, b_row):
    m = jnp.mean(x, axis=-1, keepdims=True)
    v = jnp.mean((x - m) ** 2, axis=-1, keepdims=True)
    return (x - m) / jnp.sqrt(v + 1e-5) * g_row + b_row


def _body(val_ref, ts_ref, pad_ref, var_ref,
          wencT_ref, bencT_ref, cls_row_ref,
          e1a_ref, e2Ta_ref, wsTa_ref, wmTa_ref,
          blTa_ref, bla_row_ref, gTa_ref, ga_row_ref, bbTa_ref, bba_row_ref,
          e1b_ref, e2Tb_ref, wsTb_ref, wmTb_ref,
          blTb_ref, blb_row_ref, gTb_ref, gb_row_ref, bbTb_ref, bbb_row_ref,
          w1T_ref, b1_row_ref, w2T_ref, b2_row_ref,
          out_ref, a_scr):
    b = pl.program_id(0)

    f32 = jnp.float32
    dot = lambda x, y: jax.lax.dot_general(
        x, y, (((1,), (0,)), ((), ())), preferred_element_type=f32)
    dot_t = lambda x, y: jax.lax.dot_general(
        x, y, (((1,), (1,)), ((), ())), preferred_element_type=f32)

    @pl.when(b == 0)
    def _():
        a1 = _softmax_rows(jnp.maximum(dot(e1a_ref[...], e2Ta_ref[...]), 0.0))
        a2 = _softmax_rows(jnp.maximum(dot(e1b_ref[...], e2Tb_ref[...]), 0.0))
        a_scr[0] = a1
        a_scr[1] = a2

    def _one(val, ts, pad, var):
        d_iota = jax.lax.broadcasted_iota(jnp.int32, (_HID, 1), 0)
        k = jnp.remainder(d_iota, _HALF).astype(f32)
        freqs_t = jnp.exp(-k * (np.log(10000.0) / _HALF)) * np.float32(
            1.0 / (2.0 * np.pi))
        phase = jnp.where(d_iota < _HALF, 0.0, 0.25).astype(f32)
        u = ts * freqs_t + phase
        r = u - jnp.floor(u + 0.5)
        r2 = r * r
        c1, c3, c5, c7, c9, c11 = (6.283183465946354, -41.341480313261826,
                                   81.59765670699092, -76.59491552318971,
                                   41.26987033307261, -12.372395737093948)
        s = c9 + r2 * c11
        s = c7 + r2 * s
        s = c5 + r2 * s
        s = c3 + r2 * s
        s = c1 + r2 * s
        temb = r * s
        h0 = val * wencT_ref[...] + bencT_ref[...] + temb

        n_iota = jax.lax.broadcasted_iota(jnp.int32, (_NEXT, _L), 0)
        ohT = (n_iota == var).astype(f32)

        a1 = a_scr[0]
        hp = h0 * pad
        big = jnp.concatenate([hp, pad, ohT], axis=0)
        s1c = dot_t(big[:_HID + 1], ohT)
        s1 = s1c[:_HID]
        cnt = s1c[_HID:_HID + 1]
        inv_cnt = 1.0 / jnp.maximum(cnt, 1.0)
        v1 = s1 * inv_cnt
        m1 = dot_t(v1, a1)
        mw1 = dot(wmTa_ref[...], m1)
        zlhs = jnp.concatenate(
            [wsTa_ref[...], jnp.zeros((_HID, 1), f32), mw1], axis=1)
        z = dot(zlhs, big) + blTa_ref[...]
        x = hp + jnp.maximum(z, 0.0)
        ones_r = jnp.full((1, _HID), 1.0 / _HID, f32)
        mu = dot(ones_r, x)
        ms = dot(ones_r, x * x)
        sig = ms - mu * mu
        w = pad / jnp.sqrt(sig + 1e-5)
        y = (x - mu) * w

        hc0 = cls_row_ref[...]
        m1_cls = dot_t(a1[_NN:_NN + 1, :], v1)
        zc = (dot_t(hc0, wsTa_ref[...]) + dot_t(m1_cls, wmTa_ref[...])
              + bla_row_ref[...])
        hc1 = _ln_lanes(hc0 + jnp.maximum(zc, 0.0),
                        ga_row_ref[...], bba_row_ref[...])

        a2 = a_scr[1]
        s2 = gTa_ref[...] * dot_t(y, ohT) + bbTa_ref[...] * cnt
        v2 = s2 * inv_cnt
        m2_cls = dot_t(a2[_NN:_NN + 1, :], v2)
        zc2 = (dot_t(hc1, wsTb_ref[...]) + dot_t(m2_cls, wmTb_ref[...])
               + blb_row_ref[...])
        hc2 = _ln_lanes(hc1 + jnp.maximum(zc2, 0.0),
                        gb_row_ref[...], bbb_row_ref[...])

        t = jnp.maximum(dot_t(hc2, w1T_ref[...]) + b1_row_ref[...], 0.0)
        return dot_t(t, w2T_ref[...]) + b2_row_ref[...]

    outs = [_one(val_ref[0, i:i + 1], ts_ref[0, i:i + 1],
                 pad_ref[0, i:i + 1], var_ref[0, i:i + 1])
            for i in range(_MB)]
    out_ref[0] = jnp.concatenate(outs, axis=0)


def kernel(batch_value, batch_timestamp, batch_pad_mask,
           W_enc, b_enc, class_token,
           E1_0, E2_0, Ws_0, Wm_0, bl_0, g_0, bb_0,
           E1_1, E2_1, Ws_1, Wm_1, bl_1, g_1, bb_1,
           W1, b1, W2, b2, batch_var_idx):
    f32 = jnp.float32
    col = lambda v: v.reshape(_HID, 1).astype(f32)
    row = lambda v: v.reshape(1, -1).astype(f32)

    nb = _B // _MB
    b3 = lambda a: a.reshape(nb, _MB, _L)
    operands = (
        b3(batch_value), b3(batch_timestamp), b3(batch_pad_mask), b3(batch_var_idx),
        W_enc.reshape(_HID, 1), col(b_enc), class_token.reshape(1, _HID),
        E1_0, E2_0.T, Ws_0.T, Wm_0.T,
        col(bl_0), row(bl_0), col(g_0), row(g_0), col(bb_0), row(bb_0),
        E1_1, E2_1.T, Ws_1.T, Wm_1.T,
        col(bl_1), row(bl_1), col(g_1), row(g_1), col(bb_1), row(bb_1),
        W1.T, row(b1), W2.T, row(b2),
    )

    bl_spec = pl.BlockSpec((1, _MB, _L), lambda b: (b, 0, 0))
    full = lambda a: pl.BlockSpec(a.shape, lambda b: (0,) * a.ndim)
    in_specs = [bl_spec, bl_spec, bl_spec, bl_spec] + [full(a) for a in operands[4:]]

    out = pl.pallas_call(
        _body,
        grid=(nb,),
        in_specs=in_specs,
        out_specs=pl.BlockSpec((1, _MB, _COUT), lambda b: (b, 0, 0)),
        out_shape=jax.ShapeDtypeStruct((nb, _MB, _COUT), f32),
        scratch_shapes=[pltpu.VMEM((2, _NEXT, _NEXT), f32)],
        compiler_params=pltpu.CompilerParams(
            dimension_semantics=("arbitrary",),
        ),
    )(*operands)
    return out.reshape(_B, _COUT)


# speedup vs baseline: 1.0329x; 1.0329x over previous
import numpy as np

import jax
import jax.numpy as jnp
from jax.experimental import pallas as pl
from jax.experimental.pallas import tpu as pltpu

_B, _L, _HID, _NN = 64, 4096, 64, 128
_NEXT = _NN + 1
_ND, _COUT = 32, 10
_HALF = _HID // 2
_MB = 8


def _softmax_rows(x):
    z = x - jnp.max(x, axis=-1, keepdims=True)
    e = jnp.exp(z)
    return e / jnp.sum(e, axis=-1, keepdims=True)


def _ln_lanes(x, g_row, b_row):
    m = jnp.mean(x, axis=-1, keepdims=True)
    v = jnp.mean((x - m) ** 2, axis=-1, keepdims=True)
    return (x - m) / jnp.sqrt(v + 1e-5) * g_row + b_row


def _body(val_ref, ts_ref, pad_ref, var_ref,
          wencT_ref, bencT_ref, cls_row_ref,
          e1a_ref, e2Ta_ref, wsTa_ref, wmTa_ref,
          blTa_ref, bla_row_ref, gTa_ref, ga_row_ref, bbTa_ref, bba_row_ref,
          e1b_ref, e2Tb_ref, wsTb_ref, wmTb_ref,
          blTb_ref, blb_row_ref, gTb_ref, gb_row_ref, bbTb_ref, bbb_row_ref,
          w1T_ref, b1_row_ref, w2T_ref, b2_row_ref,
          out_ref, a_scr):
    b = pl.program_id(0)

    f32 = jnp.float32
    dot = lambda x, y: jax.lax.dot_general(
        x, y, (((1,), (0,)), ((), ())), preferred_element_type=f32)
    dot_t = lambda x, y: jax.lax.dot_general(
        x, y, (((1,), (1,)), ((), ())), preferred_element_type=f32)

    @pl.when(b == 0)
    def _():
        a1 = _softmax_rows(jnp.maximum(dot(e1a_ref[...], e2Ta_ref[...]), 0.0))
        a2 = _softmax_rows(jnp.maximum(dot(e1b_ref[...], e2Tb_ref[...]), 0.0))
        a_scr[0] = a1
        a_scr[1] = a2

    def _one(val, ts, pad, var):
        d_iota = jax.lax.broadcasted_iota(jnp.int32, (_HID, 1), 0)
        k = jnp.remainder(d_iota, _HALF).astype(f32)
        freqs_t = jnp.exp(-k * (np.log(10000.0) / _HALF)) * np.float32(
            1.0 / (2.0 * np.pi))
        phase = jnp.where(d_iota < _HALF, 0.0, 0.25).astype(f32)
        u = ts * freqs_t + phase
        r = u - jnp.floor(u + 0.5)
        r2 = r * r
        c1, c3, c5, c7, c9, c11 = (6.283183465946354, -41.341480313261826,
                                   81.59765670699092, -76.59491552318971,
                                   41.26987033307261, -12.372395737093948)
        s = c9 + r2 * c11
        s = c7 + r2 * s
        s = c5 + r2 * s
        s = c3 + r2 * s
        s = c1 + r2 * s
        temb = r * s
        h0 = val * wencT_ref[...] + bencT_ref[...] + temb

        n_iota = jax.lax.broadcasted_iota(jnp.int32, (_NEXT, _L), 0)
        ohT = (n_iota == var).astype(f32)

        a1 = a_scr[0]
        hp = h0 * pad
        s1c = dot_t(jnp.concatenate([hp, pad], axis=0), ohT)
        s1 = s1c[:_HID]
        cnt = s1c[_HID:_HID + 1]
        inv_cnt = 1.0 / jnp.maximum(cnt, 1.0)
        v1 = s1 * inv_cnt
        m1 = dot_t(v1, a1)
        mw1 = dot(wmTa_ref[...], m1)
        zlhs = jnp.concatenate([wsTa_ref[...], mw1], axis=1)
        z = dot(zlhs, jnp.concatenate([hp, ohT], axis=0)) + blTa_ref[...]
        x = hp + jnp.maximum(z, 0.0)
        ones_r = jnp.full((1, _HID), 1.0 / _HID, f32)
        mu = dot(ones_r, x)
        ms = dot(ones_r, x * x)
        sig = ms - mu * mu
        w = pad / jnp.sqrt(sig + 1e-5)
        y = (x - mu) * w

        hc0 = cls_row_ref[...]
        m1_cls = dot_t(a1[_NN:_NN + 1, :], v1)
        zc = (dot_t(hc0, wsTa_ref[...]) + dot_t(m1_cls, wmTa_ref[...])
              + bla_row_ref[...])
        hc1 = _ln_lanes(hc0 + jnp.maximum(zc, 0.0),
                        ga_row_ref[...], bba_row_ref[...])

        a2 = a_scr[1]
        s2 = gTa_ref[...] * dot_t(y, ohT) + bbTa_ref[...] * cnt
        v2 = s2 * inv_cnt
        m2_cls = dot_t(a2[_NN:_NN + 1, :], v2)
        zc2 = (dot_t(hc1, wsTb_ref[...]) + dot_t(m2_cls, wmTb_ref[...])
               + blb_row_ref[...])
        hc2 = _ln_lanes(hc1 + jnp.maximum(zc2, 0.0),
                        gb_row_ref[...], bbb_row_ref[...])

        t = jnp.maximum(dot_t(hc2, w1T_ref[...]) + b1_row_ref[...], 0.0)
        return dot_t(t, w2T_ref[...]) + b2_row_ref[...]

    outs = [_one(val_ref[0, i:i + 1], ts_ref[0, i:i + 1],
                 pad_ref[0, i:i + 1], var_ref[0, i:i + 1])
            for i in range(_MB)]
    out_ref[0] = jnp.concatenate(outs, axis=0)


def kernel(batch_value, batch_timestamp, batch_pad_mask,
           W_enc, b_enc, class_token,
           E1_0, E2_0, Ws_0, Wm_0, bl_0, g_0, bb_0,
           E1_1, E2_1, Ws_1, Wm_1, bl_1, g_1, bb_1,
           W1, b1, W2, b2, batch_var_idx):
    f32 = jnp.float32
    col = lambda v: v.reshape(_HID, 1).astype(f32)
    row = lambda v: v.reshape(1, -1).astype(f32)

    nb = _B // _MB
    b3 = lambda a: a.reshape(nb, _MB, _L)
    operands = (
        b3(batch_value), b3(batch_timestamp), b3(batch_pad_mask), b3(batch_var_idx),
        W_enc.reshape(_HID, 1), col(b_enc), class_token.reshape(1, _HID),
        E1_0, E2_0.T, Ws_0.T, Wm_0.T,
        col(bl_0), row(bl_0), col(g_0), row(g_0), col(bb_0), row(bb_0),
        E1_1, E2_1.T, Ws_1.T, Wm_1.T,
        col(bl_1), row(bl_1), col(g_1), row(g_1), col(bb_1), row(bb_1),
        W1.T, row(b1), W2.T, row(b2),
    )

    bl_spec = pl.BlockSpec((1, _MB, _L), lambda b: (b, 0, 0))
    full = lambda a: pl.BlockSpec(a.shape, lambda b: (0,) * a.ndim)
    in_specs = [bl_spec, bl_spec, bl_spec, bl_spec] + [full(a) for a in operands[4:]]

    out = pl.pallas_call(
        _body,
        grid=(nb,),
        in_specs=in_specs,
        out_specs=pl.BlockSpec((1, _MB, _COUT), lambda b: (b, 0, 0)),
        out_shape=jax.ShapeDtypeStruct((nb, _MB, _COUT), f32),
        scratch_shapes=[pltpu.VMEM((2, _NEXT, _NEXT), f32)],
        compiler_params=pltpu.CompilerParams(
            dimension_semantics=("arbitrary",),
        ),
    )(*operands)
    return out.reshape(_B, _COUT)
